# baseline (speedup 1.0000x reference)
import functools

import jax
import jax.numpy as jnp
from jax import lax
from jax.experimental import pallas as pl
from jax.experimental.pallas import tpu as pltpu

_VMEM_LIMIT = 48 * 1024 * 1024


def _cp(*sem):
    return pltpu.CompilerParams(dimension_semantics=sem,
                                vmem_limit_bytes=_VMEM_LIMIT)


def _conv_kernel(*refs, B, H, W, C, fused_prologue):
    if fused_prologue:
        x_ref, w_ref, scale_ref, shift_ref, y_ref, stats_ref, xpad, patch = refs
    else:
        x_ref, w_ref, y_ref, stats_ref, xpad, patch = refs
        scale_ref = shift_ref = None
    Hp, Wp = H + 2, W + 2
    P = H * W

    xpad[0:1, :, :] = jnp.zeros((1, Wp, C), xpad.dtype)
    xpad[Hp - 1:Hp, :, :] = jnp.zeros((1, Wp, C), xpad.dtype)
    xpad[:, 0:1, :] = jnp.zeros((Hp, 1, C), xpad.dtype)
    xpad[:, Wp - 1:Wp, :] = jnp.zeros((Hp, 1, C), xpad.dtype)

    for b in range(B):
        xin = x_ref[b]
        if fused_prologue:
            xf = xin.astype(jnp.float32) * scale_ref[...] + shift_ref[...]
            xin = jnp.maximum(xf, 0.0).astype(xpad.dtype)
        xpad[1:H + 1, 1:W + 1, :] = xin
        for kh in range(3):
            for kw in range(3):
                t = kh * 3 + kw
                patch[b * P:(b + 1) * P, t * C:(t + 1) * C] = (
                    xpad[kh:kh + H, kw:kw + W, :].reshape(P, C))

    acc = jnp.dot(patch[...], w_ref[...], preferred_element_type=jnp.float32)
    y_ref[...] = acc.reshape(B, H, W, C).astype(y_ref.dtype)
    stats_ref[0, 0:1, :] = jnp.sum(acc, axis=0, keepdims=True)
    stats_ref[0, 1:2, :] = jnp.sum(acc * acc, axis=0, keepdims=True)


def _conv3x3_bn_stats(x_nhwc, w_flat, *, block_b, prologue=None):
    N, H, W, C = x_nhwc.shape
    G = N // block_b
    in_specs = [
        pl.BlockSpec((block_b, H, W, C), lambda n: (n, 0, 0, 0)),
        pl.BlockSpec((9 * C, C), lambda n: (0, 0)),
    ]
    args = [x_nhwc, w_flat]
    if prologue is not None:
        scale, shift = prologue
        in_specs += [pl.BlockSpec((1, C), lambda n: (0, 0)),
                     pl.BlockSpec((1, C), lambda n: (0, 0))]
        args += [scale.astype(jnp.float32).reshape(1, C),
                 shift.astype(jnp.float32).reshape(1, C)]

    kern = functools.partial(_conv_kernel, B=block_b, H=H, W=W, C=C,
                             fused_prologue=prologue is not None)
    y, stats = pl.pallas_call(
        kern,
        out_shape=(jax.ShapeDtypeStruct((N, H, W, C), x_nhwc.dtype),
                   jax.ShapeDtypeStruct((G, 2, C), jnp.float32)),
        grid=(G,),
        in_specs=in_specs,
        out_specs=(pl.BlockSpec((block_b, H, W, C), lambda n: (n, 0, 0, 0)),
                   pl.BlockSpec((1, 2, C), lambda n: (n, 0, 0))),
        scratch_shapes=[
            pltpu.VMEM((H + 2, W + 2, C), x_nhwc.dtype),
            pltpu.VMEM((block_b * H * W, 9 * C), x_nhwc.dtype),
        ],
        compiler_params=_cp("parallel"),
    )(*args)
    return y, stats


def _bn_scale_shift(stats, gamma, beta, count, eps=1e-5):
    s = jnp.sum(stats, axis=0)
    mean = s[0] / count
    var = s[1] / count - mean * mean
    scale = gamma * lax.rsqrt(var + eps)
    shift = beta - mean * scale
    return scale, shift


def _finalize_kernel(y_ref, skip_ref, scale_ref, shift_ref, o_ref):
    y = y_ref[...].astype(jnp.float32)
    o = y * scale_ref[...] + shift_ref[...] + skip_ref[...].astype(jnp.float32)
    o_ref[...] = jnp.maximum(o, 0.0).astype(o_ref.dtype)


def _finalize(y2, skip, scale, shift, rows_block=4096):
    N, H, W, C = y2.shape
    rows = N * H * W
    while rows % rows_block:
        rows_block //= 2
    y_flat = y2.reshape(rows, C)
    s_flat = skip.reshape(rows, C)
    out = pl.pallas_call(
        _finalize_kernel,
        out_shape=jax.ShapeDtypeStruct((rows, C), y2.dtype),
        grid=(rows // rows_block,),
        in_specs=[pl.BlockSpec((rows_block, C), lambda i: (i, 0)),
                  pl.BlockSpec((rows_block, C), lambda i: (i, 0)),
                  pl.BlockSpec((1, C), lambda i: (0, 0)),
                  pl.BlockSpec((1, C), lambda i: (0, 0))],
        out_specs=pl.BlockSpec((rows_block, C), lambda i: (i, 0)),
        compiler_params=_cp("parallel"),
    )(y_flat, s_flat, scale.astype(jnp.float32).reshape(1, C),
      shift.astype(jnp.float32).reshape(1, C))
    return out.reshape(N, H, W, C)


def kernel(x, w1, w2, g1, beta1, g2, beta2):
    N, C, H, W = x.shape
    xh = jnp.transpose(x, (0, 2, 3, 1)).astype(jnp.bfloat16)
    w1f = w1.reshape(9 * C, C).astype(jnp.bfloat16)
    w2f = w2.reshape(9 * C, C).astype(jnp.bfloat16)

    y1, st1 = _conv3x3_bn_stats(xh, w1f, block_b=4)
    scale1, shift1 = _bn_scale_shift(st1, g1, beta1, N * H * W)

    y2, st2 = _conv3x3_bn_stats(y1, w2f, block_b=4, prologue=(scale1, shift1))
    scale2, shift2 = _bn_scale_shift(st2, g2, beta2, N * H * W)

    out = _finalize(y2, xh, scale2, shift2)
    return jnp.transpose(out, (0, 3, 1, 2)).astype(jnp.float32)


# speedup vs baseline: 1.2266x; 1.2266x over previous
import functools

import jax
import jax.numpy as jnp
from jax import lax
from jax.experimental import pallas as pl
from jax.experimental.pallas import tpu as pltpu

_VMEM_LIMIT = 48 * 1024 * 1024


def _cp(*sem):
    return pltpu.CompilerParams(dimension_semantics=sem,
                                vmem_limit_bytes=_VMEM_LIMIT)


def _conv_kernel(*refs, B, H, W, C, fused_prologue):
    if fused_prologue:
        x_ref, w_ref, scale_ref, shift_ref, y_ref, stats_ref, xpad, patch = refs
    else:
        x_ref, w_ref, y_ref, stats_ref, xpad, patch = refs
        scale_ref = shift_ref = None
    Hp, Wp = H + 2, W + 2
    P = H * W

    xpad[0:1, :, :] = jnp.zeros((1, Wp, C), xpad.dtype)
    xpad[Hp - 1:Hp, :, :] = jnp.zeros((1, Wp, C), xpad.dtype)
    xpad[:, 0:1, :] = jnp.zeros((Hp, 1, C), xpad.dtype)
    xpad[:, Wp - 1:Wp, :] = jnp.zeros((Hp, 1, C), xpad.dtype)

    for b in range(B):
        xin = x_ref[b]
        if fused_prologue:
            xf = xin.astype(jnp.float32) * scale_ref[...] + shift_ref[...]
            xin = jnp.maximum(xf, 0.0)
        xpad[1:H + 1, 1:W + 1, :] = xin.astype(xpad.dtype)
        for kh in range(3):
            for kw in range(3):
                t = kh * 3 + kw
                patch[b, :, :, t * C:(t + 1) * C] = xpad[kh:kh + H, kw:kw + W, :]

    acc = jnp.dot(patch[...].reshape(B * P, 9 * C), w_ref[...],
                  preferred_element_type=jnp.float32)
    y_ref[...] = acc.reshape(B, H, W, C).astype(y_ref.dtype)
    stats_ref[0, 0:1, :] = jnp.sum(acc, axis=0, keepdims=True)
    stats_ref[0, 1:2, :] = jnp.sum(acc * acc, axis=0, keepdims=True)


def _conv3x3_bn_stats(x_nhwc, w_flat, *, block_b, prologue=None):
    N, H, W, C = x_nhwc.shape
    G = N // block_b
    in_specs = [
        pl.BlockSpec((block_b, H, W, C), lambda n: (n, 0, 0, 0)),
        pl.BlockSpec((9 * C, C), lambda n: (0, 0)),
    ]
    args = [x_nhwc, w_flat]
    if prologue is not None:
        scale, shift = prologue
        in_specs += [pl.BlockSpec((1, C), lambda n: (0, 0)),
                     pl.BlockSpec((1, C), lambda n: (0, 0))]
        args += [scale.astype(jnp.float32).reshape(1, C),
                 shift.astype(jnp.float32).reshape(1, C)]

    kern = functools.partial(_conv_kernel, B=block_b, H=H, W=W, C=C,
                             fused_prologue=prologue is not None)
    y, stats = pl.pallas_call(
        kern,
        out_shape=(jax.ShapeDtypeStruct((N, H, W, C), jnp.bfloat16),
                   jax.ShapeDtypeStruct((G, 2, C), jnp.float32)),
        grid=(G,),
        in_specs=in_specs,
        out_specs=(pl.BlockSpec((block_b, H, W, C), lambda n: (n, 0, 0, 0)),
                   pl.BlockSpec((1, 2, C), lambda n: (n, 0, 0))),
        scratch_shapes=[
            pltpu.VMEM((H + 2, W + 2, C), jnp.bfloat16),
            pltpu.VMEM((block_b, H, W, 9 * C), jnp.bfloat16),
        ],
        compiler_params=_cp("parallel"),
    )(*args)
    return y, stats


def _bn_scale_shift(stats, gamma, beta, count, eps=1e-5):
    s = jnp.sum(stats, axis=0)
    mean = s[0] / count
    var = s[1] / count - mean * mean
    scale = gamma * lax.rsqrt(var + eps)
    shift = beta - mean * scale
    return scale, shift


def _finalize_kernel(y_ref, skip_ref, scale_ref, shift_ref, o_ref):
    y = y_ref[...].astype(jnp.float32)
    o = y * scale_ref[...] + shift_ref[...] + skip_ref[...].astype(jnp.float32)
    o_ref[...] = jnp.maximum(o, 0.0).astype(o_ref.dtype)


def _finalize(y2, skip, scale, shift, rows_block=4096):
    N, H, W, C = y2.shape
    rows = N * H * W
    while rows % rows_block:
        rows_block //= 2
    y_flat = y2.reshape(rows, C)
    s_flat = skip.reshape(rows, C)
    out = pl.pallas_call(
        _finalize_kernel,
        out_shape=jax.ShapeDtypeStruct((rows, C), jnp.float32),
        grid=(rows // rows_block,),
        in_specs=[pl.BlockSpec((rows_block, C), lambda i: (i, 0)),
                  pl.BlockSpec((rows_block, C), lambda i: (i, 0)),
                  pl.BlockSpec((1, C), lambda i: (0, 0)),
                  pl.BlockSpec((1, C), lambda i: (0, 0))],
        out_specs=pl.BlockSpec((rows_block, C), lambda i: (i, 0)),
        compiler_params=_cp("parallel"),
    )(y_flat, s_flat, scale.astype(jnp.float32).reshape(1, C),
      shift.astype(jnp.float32).reshape(1, C))
    return out.reshape(N, H, W, C)


def kernel(x, w1, w2, g1, beta1, g2, beta2):
    N, C, H, W = x.shape
    xh = jnp.transpose(x, (0, 2, 3, 1))
    w1f = w1.reshape(9 * C, C).astype(jnp.bfloat16)
    w2f = w2.reshape(9 * C, C).astype(jnp.bfloat16)

    y1, st1 = _conv3x3_bn_stats(xh, w1f, block_b=4)
    scale1, shift1 = _bn_scale_shift(st1, g1, beta1, N * H * W)

    y2, st2 = _conv3x3_bn_stats(y1, w2f, block_b=4, prologue=(scale1, shift1))
    scale2, shift2 = _bn_scale_shift(st2, g2, beta2, N * H * W)

    out = _finalize(y2, xh, scale2, shift2)
    return jnp.transpose(out, (0, 3, 1, 2))
